# baseline (speedup 1.0000x reference)
import jax
import jax.numpy as jnp
from jax.experimental import pallas as pl
from jax.experimental.pallas import tpu as pltpu


def kernel(mem, idx, val, read_idx):
    B, D = val.shape
    BM, BK = 1024, 2048
    NI, NJ = B // BM, B // BK
    SL = BM // NJ

    def body(sref, r_ref, c_ref, v_ref, mem_ref, o_ref, gbuf, sem):
        i, j = pl.program_id(0), pl.program_id(1)

        def issue_slice(blk, parity):
            def issue_one(u, carry):
                row = j * SL + u
                pltpu.make_async_copy(
                    mem_ref.at[pl.ds(sref[blk * BM + row], 1), :],
                    gbuf.at[parity, pl.ds(row, 1), :],
                    sem,
                ).start()
                return carry

            jax.lax.fori_loop(0, SL, issue_one, 0)

        @pl.when(i == 0)
        def _():
            issue_slice(0, 0)

        @pl.when(i + 1 < NI)
        def _():
            issue_slice(i + 1, (i + 1) % 2)

        r_col = r_ref[...].reshape(BM, 1)
        mask = (r_col == c_ref[...]).astype(jnp.bfloat16)
        part = jnp.dot(mask, v_ref[...], preferred_element_type=jnp.float32)

        @pl.when(j == 0)
        def _():
            o_ref[...] = part

        @pl.when(j > 0)
        def _():
            o_ref[...] += part

        @pl.when(j == NJ - 1)
        def _():
            def drain_one(u, carry):
                pltpu.make_async_copy(
                    mem_ref.at[pl.ds(0, 1), :],
                    gbuf.at[0, pl.ds(0, 1), :],
                    sem,
                ).wait()
                return carry

            jax.lax.fori_loop(0, BM, drain_one, 0)
            o_ref[...] += gbuf[i % 2]

    grid_spec = pltpu.PrefetchScalarGridSpec(
        num_scalar_prefetch=1,
        grid=(NI, NJ),
        in_specs=[
            pl.BlockSpec((1, BM), lambda i, j, s: (0, i)),
            pl.BlockSpec((1, BK), lambda i, j, s: (0, j)),
            pl.BlockSpec((BK, D), lambda i, j, s: (j, 0)),
            pl.BlockSpec(memory_space=pl.ANY),
        ],
        out_specs=pl.BlockSpec((BM, D), lambda i, j, s: (i, 0)),
        scratch_shapes=[
            pltpu.VMEM((2, BM, D), jnp.float32),
            pltpu.SemaphoreType.DMA,
        ],
    )

    return pl.pallas_call(
        body,
        grid_spec=grid_spec,
        out_shape=jax.ShapeDtypeStruct((B, D), jnp.float32),
        compiler_params=pltpu.CompilerParams(
            dimension_semantics=("arbitrary", "arbitrary"),
        ),
    )(read_idx,
      read_idx.astype(jnp.float32).reshape(1, B),
      idx.astype(jnp.float32).reshape(1, B),
      val.astype(jnp.bfloat16), mem)


# speedup vs baseline: 1.4065x; 1.4065x over previous
import jax
import jax.numpy as jnp
from jax.experimental import pallas as pl
from jax.experimental.pallas import tpu as pltpu


def kernel(mem, idx, val, read_idx):
    B, D = val.shape
    BM, BK = 1024, 2048
    NI, NJ = B // BM, B // BK
    SL = BM // NJ

    def body(sref, r_ref, c_ref, v_ref, mem_ref, o_ref, gbuf, sem):
        i, j = pl.program_id(0), pl.program_id(1)

        def issue_slice(blk, parity):
            for u in range(SL):
                row = j * SL + u
                pltpu.make_async_copy(
                    mem_ref.at[pl.ds(sref[blk * BM + row], 1), :],
                    gbuf.at[parity, pl.ds(row, 1), :],
                    sem,
                ).start()

        @pl.when(i == 0)
        def _():
            issue_slice(0, 0)

        @pl.when(i + 1 < NI)
        def _():
            issue_slice(i + 1, (i + 1) % 2)

        r_col = r_ref[...].reshape(BM, 1)
        mask = (r_col == c_ref[...]).astype(jnp.bfloat16)
        part = jnp.dot(mask, v_ref[...], preferred_element_type=jnp.float32)

        @pl.when(j == 0)
        def _():
            o_ref[...] = part

        @pl.when(j > 0)
        def _():
            o_ref[...] += part

        @pl.when(j == NJ - 1)
        def _():
            pltpu.make_async_copy(
                mem_ref.at[pl.ds(0, BM), :], gbuf.at[0], sem).wait()
            o_ref[...] += gbuf[i % 2]

    grid_spec = pltpu.PrefetchScalarGridSpec(
        num_scalar_prefetch=1,
        grid=(NI, NJ),
        in_specs=[
            pl.BlockSpec((1, BM), lambda i, j, s: (0, i)),
            pl.BlockSpec((1, BK), lambda i, j, s: (0, j)),
            pl.BlockSpec((BK, D), lambda i, j, s: (j, 0)),
            pl.BlockSpec(memory_space=pl.ANY),
        ],
        out_specs=pl.BlockSpec((BM, D), lambda i, j, s: (i, 0)),
        scratch_shapes=[
            pltpu.VMEM((2, BM, D), jnp.float32),
            pltpu.SemaphoreType.DMA,
        ],
    )

    return pl.pallas_call(
        body,
        grid_spec=grid_spec,
        out_shape=jax.ShapeDtypeStruct((B, D), jnp.float32),
        compiler_params=pltpu.CompilerParams(
            dimension_semantics=("arbitrary", "arbitrary"),
        ),
    )(read_idx,
      read_idx.astype(jnp.float32).reshape(1, B),
      idx.astype(jnp.float32).reshape(1, B),
      val.astype(jnp.bfloat16), mem)


# speedup vs baseline: 1.4579x; 1.0366x over previous
import functools

import jax
import jax.numpy as jnp
from jax import lax
from jax.experimental import pallas as pl
from jax.experimental.pallas import tpu as pltpu
from jax.experimental.pallas import tpu_sc as plsc


def _sc_gather(mem, read_idx):
    B = read_idx.shape[0]
    M, D = mem.shape
    info = plsc.get_sparse_core_info()
    NC, NS = info.num_cores, info.num_subcores
    NW = NC * NS
    b_per_w = B // NW
    K = 16
    mesh = plsc.VectorSubcoreMesh(core_axis_name="c", subcore_axis_name="s")

    @functools.partial(
        pl.kernel,
        mesh=mesh,
        out_type=jax.ShapeDtypeStruct((B, 2 * D), jnp.float32),
        scratch_types=[
            pltpu.VMEM((b_per_w,), jnp.int32),
            pltpu.VMEM((b_per_w, 2 * D), jnp.float32),
            pltpu.SemaphoreType.DMA,
        ],
        compiler_params=pltpu.CompilerParams(needs_layout_passes=False),
    )
    def gather_kernel(read_hbm, table_hbm, out_hbm, idx_v, out_v, sem):
        wid = lax.axis_index("s") * NC + lax.axis_index("c")
        base = wid * b_per_w
        pltpu.sync_copy(read_hbm.at[pl.ds(base, b_per_w)], idx_v)

        n_groups = b_per_w // K
        LOOKAHEAD = 2

        def start_group(g):
            keys = idx_v[pl.ds(g * K, K)]
            for u in range(K):
                pltpu.make_async_copy(
                    table_hbm.at[keys[u]],
                    out_v.at[g * K + u, pl.ds(0, D)], sem).start()

        for g in range(LOOKAHEAD):
            start_group(g)

        def group(g, carry):
            @pl.when(g + LOOKAHEAD < n_groups)
            def _():
                start_group(g + LOOKAHEAD)

            for _u in range(K):
                pltpu.make_async_copy(
                    table_hbm.at[0], out_v.at[0, pl.ds(0, D)], sem).wait()
            return carry

        lax.fori_loop(0, n_groups, group, 0)
        pltpu.sync_copy(out_v, out_hbm.at[pl.ds(base, b_per_w)])

    return gather_kernel(read_idx, mem)


def _tc_correction(idx, val, read_idx):
    B, D = val.shape
    BM, BK = 1024, 2048
    grid = (B // BM, B // BK)

    def body(r_ref, c_ref, v_ref, o_ref):
        j = pl.program_id(1)
        r_col = r_ref[...].reshape(BM, 1)
        mask = (r_col == c_ref[...]).astype(jnp.bfloat16)
        part = jnp.dot(mask, v_ref[...], preferred_element_type=jnp.float32)

        @pl.when(j == 0)
        def _():
            o_ref[...] = part

        @pl.when(j > 0)
        def _():
            o_ref[...] += part

    return pl.pallas_call(
        body,
        grid=grid,
        in_specs=[
            pl.BlockSpec((1, BM), lambda i, j: (0, i)),
            pl.BlockSpec((1, BK), lambda i, j: (0, j)),
            pl.BlockSpec((BK, D), lambda i, j: (j, 0)),
        ],
        out_specs=pl.BlockSpec((BM, D), lambda i, j: (i, 0)),
        out_shape=jax.ShapeDtypeStruct((B, D), jnp.float32),
        compiler_params=pltpu.CompilerParams(
            dimension_semantics=("parallel", "arbitrary"),
        ),
    )(read_idx.astype(jnp.float32).reshape(1, B),
      idx.astype(jnp.float32).reshape(1, B),
      val.astype(jnp.bfloat16))


def _tc_add(gathered2, corr):
    B, D = corr.shape
    BR = 2048
    grid = (B // BR,)

    def body(g_ref, c_ref, o_ref):
        o_ref[...] = g_ref[:, :D] + c_ref[...]

    return pl.pallas_call(
        body,
        grid=grid,
        in_specs=[
            pl.BlockSpec((BR, 2 * D), lambda i: (i, 0)),
            pl.BlockSpec((BR, D), lambda i: (i, 0)),
        ],
        out_specs=pl.BlockSpec((BR, D), lambda i: (i, 0)),
        out_shape=jax.ShapeDtypeStruct((B, D), jnp.float32),
    )(gathered2, corr)


def kernel(mem, idx, val, read_idx):
    gathered2 = _sc_gather(mem, read_idx)
    corr = _tc_correction(idx, val, read_idx)
    return _tc_add(gathered2, corr)


# speedup vs baseline: 1.5197x; 1.0424x over previous
import functools

import jax
import jax.numpy as jnp
from jax import lax
from jax.experimental import pallas as pl
from jax.experimental.pallas import tpu as pltpu
from jax.experimental.pallas import tpu_sc as plsc


def _sc_gather(mem, read_idx):
    B = read_idx.shape[0]
    M, D = mem.shape
    info = plsc.get_sparse_core_info()
    NC, NS = info.num_cores, info.num_subcores
    NW = NC * NS
    b_per_w = B // NW
    K = 16
    mesh = plsc.VectorSubcoreMesh(core_axis_name="c", subcore_axis_name="s")

    @functools.partial(
        pl.kernel,
        mesh=mesh,
        out_type=jax.ShapeDtypeStruct((B, 2 * D), jnp.float32),
        scratch_types=[
            pltpu.VMEM((b_per_w,), jnp.int32),
            pltpu.VMEM((b_per_w, 2 * D), jnp.float32),
            pltpu.SemaphoreType.DMA,
        ],
        compiler_params=pltpu.CompilerParams(needs_layout_passes=False),
    )
    def gather_kernel(read_hbm, table_hbm, out_hbm, idx_v, out_v, sem):
        wid = lax.axis_index("s") * NC + lax.axis_index("c")
        base = wid * b_per_w
        pltpu.sync_copy(read_hbm.at[pl.ds(base, b_per_w)], idx_v)

        n_groups = b_per_w // K
        LOOKAHEAD = 2

        def start_group(g):
            keys = idx_v[pl.ds(g * K, K)]
            for u in range(K):
                pltpu.make_async_copy(
                    table_hbm.at[keys[u]],
                    out_v.at[g * K + u, pl.ds(0, D)], sem).start()

        for g in range(LOOKAHEAD):
            start_group(g)

        def group(g, carry):
            @pl.when(g + LOOKAHEAD < n_groups)
            def _():
                start_group(g + LOOKAHEAD)

            for _u in range(K):
                pltpu.make_async_copy(
                    table_hbm.at[0], out_v.at[0, pl.ds(0, D)], sem).wait()
            return carry

        lax.fori_loop(0, n_groups, group, 0)
        pltpu.sync_copy(out_v, out_hbm.at[pl.ds(base, b_per_w)])

    return gather_kernel(read_idx, mem)


def _tc_correction(gathered2, idx, val, read_idx):
    B, D = val.shape
    BM, BK = 1024, 4096
    grid = (B // BM, B // BK)

    def body(r_ref, c_ref, v_ref, g_ref, o_ref):
        j = pl.program_id(1)
        r_col = r_ref[...].reshape(BM, 1)
        mask = (r_col == c_ref[...]).astype(jnp.bfloat16)
        part = jnp.dot(mask, v_ref[...], preferred_element_type=jnp.float32)

        @pl.when(j == 0)
        def _():
            o_ref[...] = g_ref[:, :D] + part

        @pl.when(j > 0)
        def _():
            o_ref[...] += part

    return pl.pallas_call(
        body,
        grid=grid,
        in_specs=[
            pl.BlockSpec((1, BM), lambda i, j: (0, i)),
            pl.BlockSpec((1, BK), lambda i, j: (0, j)),
            pl.BlockSpec((BK, D), lambda i, j: (j, 0)),
            pl.BlockSpec((BM, 2 * D), lambda i, j: (i, 0)),
        ],
        out_specs=pl.BlockSpec((BM, D), lambda i, j: (i, 0)),
        out_shape=jax.ShapeDtypeStruct((B, D), jnp.float32),
        compiler_params=pltpu.CompilerParams(
            dimension_semantics=("parallel", "arbitrary"),
        ),
    )(read_idx.astype(jnp.float32).reshape(1, B),
      idx.astype(jnp.float32).reshape(1, B),
      val.astype(jnp.bfloat16), gathered2)


def kernel(mem, idx, val, read_idx):
    gathered2 = _sc_gather(mem, read_idx)
    return _tc_correction(gathered2, idx, val, read_idx)


# speedup vs baseline: 1.5371x; 1.0115x over previous
import functools

import jax
import jax.numpy as jnp
from jax import lax
from jax.experimental import pallas as pl
from jax.experimental.pallas import tpu as pltpu
from jax.experimental.pallas import tpu_sc as plsc


def _sc_gather(mem, read_idx):
    B = read_idx.shape[0]
    M, D = mem.shape
    info = plsc.get_sparse_core_info()
    NC, NS = info.num_cores, info.num_subcores
    NW = NC * NS
    b_per_w = B // NW
    K = 16
    mesh = plsc.VectorSubcoreMesh(core_axis_name="c", subcore_axis_name="s")

    @functools.partial(
        pl.kernel,
        mesh=mesh,
        out_type=jax.ShapeDtypeStruct((B, 2 * D), jnp.float32),
        scratch_types=[
            pltpu.VMEM((b_per_w,), jnp.int32),
            pltpu.VMEM((b_per_w, 2 * D), jnp.float32),
            pltpu.SemaphoreType.DMA,
        ],
        compiler_params=pltpu.CompilerParams(needs_layout_passes=False),
    )
    def gather_kernel(read_hbm, table_hbm, out_hbm, idx_v, out_v, sem):
        wid = lax.axis_index("s") * NC + lax.axis_index("c")
        base = wid * b_per_w
        pltpu.sync_copy(read_hbm.at[pl.ds(base, b_per_w)], idx_v)

        n_groups = b_per_w // K
        LOOKAHEAD = 2

        def start_group(g):
            keys = idx_v[pl.ds(g * K, K)]
            for u in range(K):
                pltpu.make_async_copy(
                    table_hbm.at[keys[u]],
                    out_v.at[g * K + u, pl.ds(0, D)], sem).start()

        for g in range(LOOKAHEAD):
            start_group(g)

        def group(g, carry):
            @pl.when(g + LOOKAHEAD < n_groups)
            def _():
                start_group(g + LOOKAHEAD)

            for _u in range(K):
                pltpu.make_async_copy(
                    table_hbm.at[0], out_v.at[0, pl.ds(0, D)], sem).wait()
            return carry

        lax.fori_loop(0, n_groups, group, 0)
        pltpu.sync_copy(out_v, out_hbm.at[pl.ds(base, b_per_w)])

    return gather_kernel(read_idx, mem)


def _tc_correction(gathered2, idx, val, read_idx):
    B, D = val.shape
    BM, BK = 2048, 4096
    grid = (B // BM, B // BK)

    def body(r_ref, c_ref, v_ref, g_ref, o_ref):
        j = pl.program_id(1)
        r_col = r_ref[...].reshape(BM, 1)
        mask = (r_col == c_ref[...]).astype(jnp.bfloat16)
        part = jnp.dot(mask, v_ref[...], preferred_element_type=jnp.float32)

        @pl.when(j == 0)
        def _():
            o_ref[...] = g_ref[:, :D] + part

        @pl.when(j > 0)
        def _():
            o_ref[...] += part

    return pl.pallas_call(
        body,
        grid=grid,
        in_specs=[
            pl.BlockSpec((1, BM), lambda i, j: (0, i)),
            pl.BlockSpec((1, BK), lambda i, j: (0, j)),
            pl.BlockSpec((BK, D), lambda i, j: (j, 0)),
            pl.BlockSpec((BM, 2 * D), lambda i, j: (i, 0)),
        ],
        out_specs=pl.BlockSpec((BM, D), lambda i, j: (i, 0)),
        out_shape=jax.ShapeDtypeStruct((B, D), jnp.float32),
        compiler_params=pltpu.CompilerParams(
            dimension_semantics=("parallel", "arbitrary"),
        ),
    )(read_idx.astype(jnp.float32).reshape(1, B),
      idx.astype(jnp.float32).reshape(1, B),
      val.astype(jnp.bfloat16), gathered2)


def kernel(mem, idx, val, read_idx):
    gathered2 = _sc_gather(mem, read_idx)
    return _tc_correction(gathered2, idx, val, read_idx)
